# Initial kernel scaffold; baseline (speedup 1.0000x reference)
import jax
import jax.numpy as jnp
from jax.experimental import pallas as pl


def kernel(x, edge_index, W, b):
    raise NotImplementedError("write your pallas kernel here")


# speedup vs baseline: 13.4794x; 13.4794x over previous
import functools

import jax
import jax.numpy as jnp
from jax import lax
from jax.experimental import pallas as pl
from jax.experimental.pallas import tpu as pltpu
from jax.experimental.pallas import tpu_sc as plsc

N_NODES = 10000
N_FEAT = 128
NC = 2
NS = 16
NW = NC * NS
CH = 128
AR = 10240
RPS = AR // NS
PAD_NODE = 10016

_sc_mesh = plsc.VectorSubcoreMesh(
    core_axis_name="c", subcore_axis_name="s", num_cores=NC, num_subcores=NS)


def _deg_body(nchunk16, col_hbm, out_hbm, hist, colv):
    c = lax.axis_index("c")
    s = lax.axis_index("s")
    w = s * NC + c
    zeros = jnp.zeros((16,), jnp.float32)

    def zero_step(i, _):
        hist[pl.ds(i * 16, 16)] = zeros
        return 0

    lax.fori_loop(0, AR // 16, zero_step, 0)
    pltpu.sync_copy(col_hbm.at[w], colv)
    ones = jnp.ones((16,), jnp.float32)

    def step(j, _):
        idx = colv[pl.ds(j * 16, 16)]
        plsc.addupdate_scatter(hist, [idx], ones)
        return 0

    lax.fori_loop(0, nchunk16, step, 0)
    pltpu.sync_copy(hist, out_hbm.at[w])


def _make_deg_kernel(epw):
    return functools.partial(
        pl.kernel,
        out_type=jax.ShapeDtypeStruct((NW, AR), jnp.float32),
        mesh=_sc_mesh,
        scratch_types=[
            pltpu.VMEM((AR,), jnp.float32),
            pltpu.VMEM((epw,), jnp.int32),
        ],
        compiler_params=pltpu.CompilerParams(needs_layout_passes=False),
    )(functools.partial(_deg_body, epw // 16))


def _prop_body(nchunk, u_hbm, row_hbm, col_hbm, out_hbm, acc, rowv, colv,
               gbuf, sem):
    c = lax.axis_index("c")
    s = lax.axis_index("s")
    w = s * NC + c
    pltpu.sync_copy(u_hbm.at[pl.ds(s * RPS, RPS)], acc.at[pl.ds(s * RPS, RPS)])
    pltpu.sync_copy(row_hbm.at[w], rowv)
    pltpu.sync_copy(col_hbm.at[w], colv)
    plsc.subcore_barrier()

    def step(j, _):
        pltpu.async_copy(u_hbm.at[colv.at[j]], gbuf, sem).wait()
        pltpu.sync_copy(gbuf, acc.at[rowv.at[j]], add=True)
        return 0

    lax.fori_loop(0, nchunk, step, 0)
    plsc.subcore_barrier()
    pltpu.sync_copy(acc.at[pl.ds(s * RPS, RPS)],
                    out_hbm.at[c].at[pl.ds(s * RPS, RPS)])


def _make_prop_kernel(nchunk):
    return functools.partial(
        pl.kernel,
        out_type=jax.ShapeDtypeStruct((NC, AR, N_FEAT), jnp.float32),
        mesh=_sc_mesh,
        scratch_types=[
            pltpu.VMEM_SHARED((AR, N_FEAT), jnp.float32),
            pltpu.VMEM((nchunk, CH), jnp.int32),
            pltpu.VMEM((nchunk, CH), jnp.int32),
            pltpu.VMEM((CH, N_FEAT), jnp.float32),
            pltpu.SemaphoreType.DMA,
        ],
    )(functools.partial(_prop_body, nchunk))


_TB = 2048


def _dis_of(hist_blk):
    deg = jnp.sum(hist_blk, axis=0) + 1.0
    return lax.rsqrt(deg)


def _prescale_body(hist_ref, x_ref, u0_ref):
    dis = _dis_of(hist_ref[...])
    u0_ref[...] = x_ref[...] * dis[:, None]


def _combine_mid_body(hist_ref, p_ref, u0_ref, u1_ref):
    dis = _dis_of(hist_ref[...])
    s1 = p_ref[0] + p_ref[1] - u0_ref[...]
    u1_ref[...] = s1 * (dis * dis)[:, None]


def _final_body(hist_ref, q_ref, u1_ref, w_ref, b_ref, out_ref):
    dis = _dis_of(hist_ref[...])
    h2 = (q_ref[0] + q_ref[1] - u1_ref[...]) * dis[:, None]
    out_ref[...] = lax.dot_general(
        h2, w_ref[...], (((1,), (1,)), ((), ())),
        preferred_element_type=jnp.float32) + b_ref[...]


def _hist_spec():
    return pl.BlockSpec((NW, _TB), lambda i: (0, i))


def _rows_spec():
    return pl.BlockSpec((_TB, N_FEAT), lambda i: (i, 0))


def _pair_spec():
    return pl.BlockSpec((NC, _TB, N_FEAT), lambda i: (0, i, 0))


def kernel(x, edge_index, W, b):
    row = edge_index[0]
    col = edge_index[1]
    e = row.shape[0]
    nchunk = -(-e // (NW * CH))
    ep = nchunk * NW * CH
    pad = ep - e
    rowp = jnp.concatenate([row, jnp.full((pad,), PAD_NODE, jnp.int32)])
    colp = jnp.concatenate([col, jnp.full((pad,), PAD_NODE, jnp.int32)])
    row3 = rowp.reshape(NW, nchunk, CH)
    col3 = colp.reshape(NW, nchunk, CH)
    colw = colp.reshape(NW, nchunk * CH)

    x_pad = jnp.zeros((AR, N_FEAT), x.dtype).at[:N_NODES].set(x)

    hists = _make_deg_kernel(nchunk * CH)(colw)

    grid = AR // _TB
    u0 = pl.pallas_call(
        _prescale_body,
        grid=(grid,),
        in_specs=[_hist_spec(), _rows_spec()],
        out_specs=_rows_spec(),
        out_shape=jax.ShapeDtypeStruct((AR, N_FEAT), jnp.float32),
    )(hists, x_pad)

    prop = _make_prop_kernel(nchunk)
    p = prop(u0, row3, col3)

    u1 = pl.pallas_call(
        _combine_mid_body,
        grid=(grid,),
        in_specs=[_hist_spec(), _pair_spec(), _rows_spec()],
        out_specs=_rows_spec(),
        out_shape=jax.ShapeDtypeStruct((AR, N_FEAT), jnp.float32),
    )(hists, p, u0)

    q = prop(u1, row3, col3)

    out = pl.pallas_call(
        _final_body,
        grid=(grid,),
        in_specs=[
            _hist_spec(), _pair_spec(), _rows_spec(),
            pl.BlockSpec((N_FEAT, N_FEAT), lambda i: (0, 0)),
            pl.BlockSpec((1, N_FEAT), lambda i: (0, 0)),
        ],
        out_specs=_rows_spec(),
        out_shape=jax.ShapeDtypeStruct((AR, N_FEAT), jnp.float32),
    )(hists, q, u1, W, b.reshape(1, N_FEAT))

    return out[:N_NODES]
